# baseline (speedup 1.0000x reference)
import functools

import jax
import jax.numpy as jnp
from jax import lax
from jax.experimental import pallas as pl
from jax.experimental.pallas import tpu as pltpu
from jax.experimental.pallas import tpu_sc as plsc

_VOCAB = 1000
_DEC = 512
_ND = 448
_NS = 64
_B = 4096
_U = 50
_YP = 112
_NDP = 512
_NSP = 128
_NR = 56
_NW = 32
_NB = _B // _NW


def _sc_decoder(ypad, wdec, wswit, ab, out, ab_v, yraw_v, idx0_v, idx1_v,
                rowsd_v, rowss_v, out_v, sem_ab, sem_y, sem_g, sem_o):
    cid = lax.axis_index("c")
    sid = lax.axis_index("s")
    wid = sid * 2 + cid
    b_base = wid * _NB

    acp = pltpu.async_copy(ab, ab_v, sem_ab)

    def start_yraw(nb, slot):
        pltpu.async_copy(ypad.at[pl.ds((b_base + nb) * _YP, _YP)],
                         yraw_v.at[slot], sem_y.at[slot])

    def wait_yraw(slot):
        pltpu.make_async_copy(ypad.at[pl.ds(0, _YP)], yraw_v.at[slot],
                              sem_y.at[slot]).wait()

    def start_gather(slot):
        pltpu.async_copy(wdec.at[idx0_v.at[slot, pl.ds(0, _NR)]],
                         rowsd_v.at[slot], sem_g.at[slot])
        pltpu.async_copy(wswit.at[idx1_v.at[slot, pl.ds(0, _NR)]],
                         rowss_v.at[slot], sem_g.at[slot])

    def wait_gather(slot):
        pltpu.make_async_copy(wdec.at[idx0_v.at[slot, pl.ds(0, _NR)]],
                              rowsd_v.at[slot], sem_g.at[slot]).wait()
        pltpu.make_async_copy(wswit.at[idx1_v.at[slot, pl.ds(0, _NR)]],
                              rowss_v.at[slot], sem_g.at[slot]).wait()

    def wait_out(slot):
        pltpu.make_async_copy(out_v.at[slot],
                              out.at[pl.ds(b_base, 1)],
                              sem_o.at[slot]).wait()

    iota = lax.iota(jnp.int32, 16)
    zero_v = jnp.zeros((16,), jnp.int32)

    def compact(slot):
        slot_v = jnp.full((16,), slot, jnp.int32)
        for k in range(4):
            pos2 = 2 * (16 * k + iota)
            g0 = plsc.load_gather(yraw_v, [slot_v, pos2])
            idx0_v[slot, pl.ds(16 * k, 16)] = g0
            g1 = plsc.load_gather(yraw_v, [slot_v, pos2 + 1])
            idx1_v[slot, pl.ds(16 * k, 16)] = g1

    start_yraw(0, 0)
    start_yraw(1, 1)
    wait_yraw(0)
    compact(0)
    start_gather(0)
    acp.wait()

    zeros = jnp.zeros((16,), jnp.float32)
    half = _U // 2

    def nb_body(nb, carry):
        slot = nb % 2
        nslot = (nb + 1) % 2

        wait_gather(slot)

        @pl.when(nb + 2 < _NB)
        def _():
            start_yraw(nb + 2, slot)

        @pl.when(nb + 1 < _NB)
        def _():
            wait_yraw(nslot)
            compact(nslot)
            start_gather(nslot)

        @pl.when(nb >= 2)
        def _():
            wait_out(slot)

        def block(rows_ref, nj, ocol):
            @plsc.parallel_loop(0, nj)
            def j_body(j):
                c = 16 * j
                a_r = ab_v[0, pl.ds(ocol + c, 16)]
                b_r = ab_v[1, pl.ds(ocol + c, 16)]
                prev_a = zeros
                prev_b = rows_ref[slot, half - 1, pl.ds(c, 16)]
                for u in range(half):
                    cur_a = rows_ref[slot, u, pl.ds(c, 16)]
                    cur_b = rows_ref[slot, half + u, pl.ds(c, 16)]
                    oa = jnp.maximum(a_r * prev_a + b_r * cur_a, 0.0)
                    ob = jnp.maximum(a_r * prev_b + b_r * cur_b, 0.0)
                    out_v[slot, 0, u, pl.ds(ocol + c, 16)] = oa
                    out_v[slot, 0, half + u, pl.ds(ocol + c, 16)] = ob
                    prev_a = cur_a
                    prev_b = cur_b

        block(rowsd_v, _ND // 16, 0)
        block(rowss_v, _NS // 16, _ND)

        pltpu.async_copy(out_v.at[slot],
                         out.at[pl.ds(b_base + nb, 1)],
                         sem_o.at[slot])
        return carry

    lax.fori_loop(0, _NB, nb_body, 0)
    wait_out(0)
    wait_out(1)


_sc_call = functools.partial(
    pl.kernel,
    mesh=plsc.VectorSubcoreMesh(core_axis_name="c", subcore_axis_name="s"),
    out_type=jax.ShapeDtypeStruct((_B, _U, _DEC), jnp.float32),
    scratch_types=[
        pltpu.VMEM((2, _DEC), jnp.float32),
        pltpu.VMEM((2, _YP), jnp.int32),
        pltpu.VMEM((2, 64), jnp.int32),
        pltpu.VMEM((2, 64), jnp.int32),
        pltpu.VMEM((2, _NR, _NDP), jnp.float32),
        pltpu.VMEM((2, _NR, _NSP), jnp.float32),
        pltpu.VMEM((2, 1, _U, _DEC), jnp.float32),
        pltpu.SemaphoreType.DMA,
        pltpu.SemaphoreType.DMA((2,)),
        pltpu.SemaphoreType.DMA((2,)),
        pltpu.SemaphoreType.DMA((2,)),
    ],
    compiler_params=pltpu.CompilerParams(needs_layout_passes=False),
)(_sc_decoder)


def kernel(y, W_dec, W_swit, s_dec, s_swit, conv_w, conv_s):
    y = y.astype(jnp.int32)
    ypad = jnp.pad(y.reshape(_B, 2 * _U), ((0, 0), (0, _YP - 2 * _U)))
    ypad = ypad.reshape(-1)
    wdec_p = jnp.pad(W_dec, ((0, 0), (0, _NDP - _ND)))
    wswit_p = jnp.pad(W_swit, ((0, 0), (0, _NSP - _NS)))
    escale = jnp.concatenate([
        jnp.full((_ND,), jnp.exp(s_dec), jnp.float32),
        jnp.full((_NS,), jnp.exp(s_swit), jnp.float32),
    ])
    wscale = jnp.exp(conv_s) * escale
    a_tap = conv_w[:, 0, 0] * wscale
    b_tap = conv_w[:, 0, 1] * wscale
    ab = jnp.stack([a_tap, b_tap], 0)
    return _sc_call(ypad, wdec_p, wswit_p, ab)


# speedup vs baseline: 1.9703x; 1.9703x over previous
import functools

import jax
import jax.numpy as jnp
from jax import lax
from jax.experimental import pallas as pl
from jax.experimental.pallas import tpu as pltpu
from jax.experimental.pallas import tpu_sc as plsc

_VOCAB = 1000
_DEC = 512
_NFEAT_DEC = 448
_B = 4096
_U = 50
_NCHUNK = 8
_CW = 64
_NGROUP = 4
_GB = _B // _NGROUP
_BBLK = 8
_NROW = _BBLK * _U
_NB = _GB // _BBLK


def _sc_decoder(t8, ab8, i0, i1, out, ab_v, idx_v, rows_v, out_v,
                sem_ab, sem_i, sem_g, sem_o):
    cid = lax.axis_index("c")
    sid = lax.axis_index("s")
    wid = sid * 2 + cid
    chunk = wid % _NCHUNK
    bg = wid // _NCHUNK
    b_base = bg * _GB
    is_dec = chunk < _NCHUNK - 1

    acp = pltpu.async_copy(ab8.at[chunk], ab_v, sem_ab)

    def start_idx(nb, slot):
        r0 = (b_base + nb * _BBLK) * _U

        @pl.when(is_dec)
        def _():
            pltpu.async_copy(i0.at[pl.ds(r0, _NROW)], idx_v.at[slot],
                             sem_i.at[slot])

        @pl.when(jnp.logical_not(is_dec))
        def _():
            pltpu.async_copy(i1.at[pl.ds(r0, _NROW)], idx_v.at[slot],
                             sem_i.at[slot])

    def wait_idx(slot):
        pltpu.make_async_copy(i0.at[pl.ds(0, _NROW)], idx_v.at[slot],
                              sem_i.at[slot]).wait()

    def start_gather(slot):
        pltpu.async_copy(t8.at[chunk].at[idx_v.at[slot]], rows_v.at[slot],
                         sem_g.at[slot])

    def wait_gather(slot):
        pltpu.make_async_copy(t8.at[0].at[idx_v.at[slot]], rows_v.at[slot],
                              sem_g.at[slot]).wait()

    def wait_out(slot):
        pltpu.make_async_copy(
            out_v.at[slot],
            out.at[pl.ds(b_base, _BBLK), :, pl.ds(chunk * _CW, _CW)],
            sem_o.at[slot]).wait()

    start_idx(0, 0)
    start_idx(1, 1)
    wait_idx(0)
    start_gather(0)
    acp.wait()

    a_regs = [ab_v[0, pl.ds(16 * s, 16)] for s in range(4)]
    b_regs = [ab_v[1, pl.ds(16 * s, 16)] for s in range(4)]
    zeros = jnp.zeros((16,), jnp.float32)
    half = _U // 2

    def nb_body(nb, carry):
        slot = nb % 2
        nslot = (nb + 1) % 2

        wait_gather(slot)

        @pl.when(nb + 2 < _NB)
        def _():
            start_idx(nb + 2, slot)

        @pl.when(nb + 1 < _NB)
        def _():
            wait_idx(nslot)
            start_gather(nslot)

        @pl.when(nb >= 2)
        def _():
            wait_out(slot)

        @plsc.parallel_loop(0, _BBLK)
        def bi_body(bi):
            r = bi * _U
            prev_a = [zeros, zeros, zeros, zeros]
            prev_b = [rows_v[slot, r + half - 1, pl.ds(16 * s, 16)]
                      for s in range(4)]
            for u in range(half):
                cur_a = [rows_v[slot, r + u, pl.ds(16 * s, 16)]
                         for s in range(4)]
                cur_b = [rows_v[slot, r + half + u, pl.ds(16 * s, 16)]
                         for s in range(4)]
                oa = [jnp.maximum(a_regs[s] * prev_a[s]
                                  + b_regs[s] * cur_a[s], 0.0)
                      for s in range(4)]
                ob = [jnp.maximum(a_regs[s] * prev_b[s]
                                  + b_regs[s] * cur_b[s], 0.0)
                      for s in range(4)]
                for s in range(4):
                    out_v[slot, bi, u, pl.ds(16 * s, 16)] = oa[s]
                for s in range(4):
                    out_v[slot, bi, half + u, pl.ds(16 * s, 16)] = ob[s]
                prev_a = cur_a
                prev_b = cur_b

        b0 = b_base + nb * _BBLK
        pltpu.async_copy(
            out_v.at[slot],
            out.at[pl.ds(b0, _BBLK), :, pl.ds(chunk * _CW, _CW)],
            sem_o.at[slot])
        return carry

    lax.fori_loop(0, _NB, nb_body, 0)
    wait_out(0)
    wait_out(1)


_sc_call = functools.partial(
    pl.kernel,
    mesh=plsc.VectorSubcoreMesh(core_axis_name="c", subcore_axis_name="s"),
    out_type=jax.ShapeDtypeStruct((_B, _U, _DEC), jnp.float32),
    scratch_types=[
        pltpu.VMEM((2, _CW), jnp.float32),
        pltpu.VMEM((2, _NROW), jnp.int32),
        pltpu.VMEM((2, _NROW, _CW), jnp.float32),
        pltpu.VMEM((2, _BBLK, _U, _CW), jnp.float32),
        pltpu.SemaphoreType.DMA,
        pltpu.SemaphoreType.DMA((2,)),
        pltpu.SemaphoreType.DMA((2,)),
        pltpu.SemaphoreType.DMA((2,)),
    ],
    compiler_params=pltpu.CompilerParams(use_tc_tiling_on_sc=False,
                                         needs_layout_passes=False),
)(_sc_decoder)


def kernel(y, W_dec, W_swit, s_dec, s_swit, conv_w, conv_s):
    y = y.astype(jnp.int32)
    idx0 = y[:, :, 0].reshape(-1)
    idx1 = y[:, :, 1].reshape(-1)
    table = jnp.concatenate([W_dec, W_swit], axis=1)
    t8 = table.reshape(_VOCAB, _NCHUNK, _CW).transpose(1, 0, 2)
    escale = jnp.concatenate([
        jnp.full((_NFEAT_DEC,), jnp.exp(s_dec), jnp.float32),
        jnp.full((_DEC - _NFEAT_DEC,), jnp.exp(s_swit), jnp.float32),
    ])
    wscale = jnp.exp(conv_s) * escale
    a_tap = conv_w[:, 0, 0] * wscale
    b_tap = conv_w[:, 0, 1] * wscale
    ab8 = jnp.stack([a_tap, b_tap], 0).reshape(2, _NCHUNK, _CW)
    ab8 = ab8.transpose(1, 0, 2)
    return _sc_call(t8, ab8, idx0, idx1)


# speedup vs baseline: 1.9770x; 1.0034x over previous
import functools

import jax
import jax.numpy as jnp
from jax import lax
from jax.experimental import pallas as pl
from jax.experimental.pallas import tpu as pltpu
from jax.experimental.pallas import tpu_sc as plsc

_VOCAB = 1000
_DEC = 512
_NFEAT_DEC = 448
_B = 4096
_U = 50
_NCHUNK = 8
_CW = 64
_NGROUP = 4
_GB = _B // _NGROUP
_BBLK = 8
_NROW = _BBLK * _U
_NB = _GB // _BBLK


def _sc_decoder(t8, ab8, i0, i1, out, ab_v, idx_v, rows_v, out_v,
                sem_ab, sem_i, sem_g, sem_o):
    cid = lax.axis_index("c")
    sid = lax.axis_index("s")
    wid = sid * 2 + cid
    chunk = wid % _NCHUNK
    bg = wid // _NCHUNK
    b_base = bg * _GB
    is_dec = chunk < _NCHUNK - 1

    acp = pltpu.async_copy(ab8.at[chunk], ab_v, sem_ab)

    def start_idx(nb, slot):
        r0 = (b_base + nb * _BBLK) * _U

        @pl.when(is_dec)
        def _():
            pltpu.async_copy(i0.at[pl.ds(r0, _NROW)], idx_v.at[slot],
                             sem_i.at[slot])

        @pl.when(jnp.logical_not(is_dec))
        def _():
            pltpu.async_copy(i1.at[pl.ds(r0, _NROW)], idx_v.at[slot],
                             sem_i.at[slot])

    def wait_idx(slot):
        pltpu.make_async_copy(i0.at[pl.ds(0, _NROW)], idx_v.at[slot],
                              sem_i.at[slot]).wait()

    def start_gather(slot):
        pltpu.async_copy(t8.at[chunk].at[idx_v.at[slot]], rows_v.at[slot],
                         sem_g.at[slot])

    def wait_gather(slot):
        pltpu.make_async_copy(t8.at[0].at[idx_v.at[slot]], rows_v.at[slot],
                              sem_g.at[slot]).wait()

    def wait_out(slot):
        pltpu.make_async_copy(
            out_v.at[slot],
            out.at[pl.ds(b_base, _BBLK), :, pl.ds(chunk * _CW, _CW)],
            sem_o.at[slot]).wait()

    start_idx(0, 0)
    start_idx(1, 1)
    wait_idx(0)
    start_gather(0)
    acp.wait()

    a_regs = [ab_v[0, pl.ds(16 * s, 16)] for s in range(4)]
    b_regs = [ab_v[1, pl.ds(16 * s, 16)] for s in range(4)]
    zeros = jnp.zeros((16,), jnp.float32)
    half = _U // 2

    def nb_body(nb, carry):
        slot = nb % 2
        nslot = (nb + 1) % 2

        wait_gather(slot)

        @pl.when(nb + 2 < _NB)
        def _():
            start_idx(nb + 2, slot)

        @pl.when(nb + 1 < _NB)
        def _():
            wait_idx(nslot)
            start_gather(nslot)

        @pl.when(nb >= 2)
        def _():
            wait_out(slot)

        starts = (0, 13, 25, 38)
        lens = (13, 12, 13, 12)

        @plsc.parallel_loop(0, _BBLK)
        def bi_body(bi):
            r = bi * _U
            prevs = [[zeros] * 4 if q == 0 else
                     [rows_v[slot, r + starts[q] - 1, pl.ds(16 * s, 16)]
                      for s in range(4)]
                     for q in range(4)]
            for i in range(13):
                qs = [q for q in range(4) if i < lens[q]]
                curs = {q: [rows_v[slot, r + starts[q] + i,
                                   pl.ds(16 * s, 16)]
                            for s in range(4)] for q in qs}
                outs = {q: [jnp.maximum(a_regs[s] * prevs[q][s]
                                        + b_regs[s] * curs[q][s], 0.0)
                            for s in range(4)] for q in qs}
                for q in qs:
                    for s in range(4):
                        out_v[slot, bi, starts[q] + i,
                              pl.ds(16 * s, 16)] = outs[q][s]
                for q in qs:
                    prevs[q] = curs[q]

        b0 = b_base + nb * _BBLK
        pltpu.async_copy(
            out_v.at[slot],
            out.at[pl.ds(b0, _BBLK), :, pl.ds(chunk * _CW, _CW)],
            sem_o.at[slot])
        return carry

    lax.fori_loop(0, _NB, nb_body, 0)
    wait_out(0)
    wait_out(1)


_sc_call = functools.partial(
    pl.kernel,
    mesh=plsc.VectorSubcoreMesh(core_axis_name="c", subcore_axis_name="s"),
    out_type=jax.ShapeDtypeStruct((_B, _U, _DEC), jnp.float32),
    scratch_types=[
        pltpu.VMEM((2, _CW), jnp.float32),
        pltpu.VMEM((2, _NROW), jnp.int32),
        pltpu.VMEM((2, _NROW, _CW), jnp.float32),
        pltpu.VMEM((2, _BBLK, _U, _CW), jnp.float32),
        pltpu.SemaphoreType.DMA,
        pltpu.SemaphoreType.DMA((2,)),
        pltpu.SemaphoreType.DMA((2,)),
        pltpu.SemaphoreType.DMA((2,)),
    ],
    compiler_params=pltpu.CompilerParams(use_tc_tiling_on_sc=False,
                                         needs_layout_passes=False),
)(_sc_decoder)


def kernel(y, W_dec, W_swit, s_dec, s_swit, conv_w, conv_s):
    y = y.astype(jnp.int32)
    idx0 = y[:, :, 0].reshape(-1)
    idx1 = y[:, :, 1].reshape(-1)
    table = jnp.concatenate([W_dec, W_swit], axis=1)
    t8 = table.reshape(_VOCAB, _NCHUNK, _CW).transpose(1, 0, 2)
    escale = jnp.concatenate([
        jnp.full((_NFEAT_DEC,), jnp.exp(s_dec), jnp.float32),
        jnp.full((_DEC - _NFEAT_DEC,), jnp.exp(s_swit), jnp.float32),
    ])
    wscale = jnp.exp(conv_s) * escale
    a_tap = conv_w[:, 0, 0] * wscale
    b_tap = conv_w[:, 0, 1] * wscale
    ab8 = jnp.stack([a_tap, b_tap], 0).reshape(2, _NCHUNK, _CW)
    ab8 = ab8.transpose(1, 0, 2)
    return _sc_call(t8, ab8, idx0, idx1)
